# baseline (speedup 1.0000x reference)
import functools

import jax
import jax.numpy as jnp
from jax import lax
from jax.experimental import pallas as pl
from jax.experimental.pallas import tpu as pltpu
from jax.experimental.pallas import tpu_sc as plsc


def _expert_body(x_ref, w1_ref, b1_ref, w2_ref, b2_ref, g_ref,
                 out_ref, diag_ref, acc_o, acc_s1, acc_s2, *, n_e):
    e = pl.program_id(1)

    h = jnp.dot(x_ref[...], w1_ref[0], preferred_element_type=jnp.float32)
    h = jnp.maximum(h + b1_ref[0], 0.0)
    ex = jnp.dot(h.astype(jnp.bfloat16), w2_ref[0],
                 preferred_element_type=jnp.float32)
    ex = ex + b2_ref[0]
    sel = jax.lax.broadcasted_iota(jnp.int32, (1, n_e), 1) == e
    g = jnp.sum(jnp.where(sel, g_ref[...], 0.0), axis=1, keepdims=True)

    @pl.when(e == 0)
    def _():
        acc_o[...] = ex * g
        acc_s1[...] = ex
        acc_s2[...] = ex * ex

    @pl.when(e > 0)
    def _():
        acc_o[...] += ex * g
        acc_s1[...] += ex
        acc_s2[...] += ex * ex

    @pl.when(e == n_e - 1)
    def _():
        out_ref[...] = acc_o[...]
        s1 = acc_s1[...]
        diag_ref[...] = (acc_s2[...] - s1 * s1 * (1.0 / n_e)) * (1.0 / (n_e - 1))


def _experts_and_combine(xf, w1bf, b1, w2bf, b2, gate, *, interpret=False):
    Bn, CD = xf.shape
    E, _, H = w1bf.shape
    K = w2bf.shape[2]
    BT = min(512, Bn)
    grid = (Bn // BT, E)
    return pl.pallas_call(
        functools.partial(_expert_body, n_e=E),
        grid=grid,
        in_specs=[
            pl.BlockSpec((BT, CD), lambda i, e: (i, 0)),
            pl.BlockSpec((1, CD, H), lambda i, e: (e, 0, 0)),
            pl.BlockSpec((1, 1, H), lambda i, e: (e, 0, 0)),
            pl.BlockSpec((1, H, K), lambda i, e: (e, 0, 0)),
            pl.BlockSpec((1, 1, K), lambda i, e: (e, 0, 0)),
            pl.BlockSpec((BT, E), lambda i, e: (i, 0)),
        ],
        out_specs=[
            pl.BlockSpec((BT, K), lambda i, e: (i, 0)),
            pl.BlockSpec((BT, K), lambda i, e: (i, 0)),
        ],
        out_shape=[
            jax.ShapeDtypeStruct((Bn, K), jnp.float32),
            jax.ShapeDtypeStruct((Bn, K), jnp.float32),
        ],
        scratch_shapes=[
            pltpu.VMEM((BT, K), jnp.float32),
            pltpu.VMEM((BT, K), jnp.float32),
            pltpu.VMEM((BT, K), jnp.float32),
        ],
        compiler_params=pltpu.CompilerParams(
            dimension_semantics=("parallel", "arbitrary"),
            vmem_limit_bytes=63 * 1024 * 1024,
        ),
        interpret=interpret,
    )(xf, w1bf, b1[:, None, :], w2bf, b2[:, None, :], gate)


def _gate_body(x_ref, wg_ref, bg_ref, out_ref, xbf_ref, *, c):
    xb = x_ref[...]
    xbf_ref[...] = xb.astype(jnp.bfloat16)
    gl = jnp.dot(xb, wg_ref[...],
                 preferred_element_type=jnp.float32) + bg_ref[...]
    m = jnp.max(gl, axis=1, keepdims=True)
    p = jnp.exp(gl - m)
    g = p / jnp.sum(p, axis=1, keepdims=True)
    bt, n_e = out_ref.shape
    out_ref[...] = jnp.sum(g.reshape(bt, c, n_e), axis=1) * (1.0 / c)


def _gate_tc(x2, wg, bg, bn, c):
    d = x2.shape[1]
    n_e = wg.shape[1]
    BT = 256
    return pl.pallas_call(
        functools.partial(_gate_body, c=c),
        grid=(bn // BT,),
        in_specs=[
            pl.BlockSpec((BT * c, d), lambda i: (i, 0)),
            pl.BlockSpec((d, n_e), lambda i: (0, 0)),
            pl.BlockSpec((1, n_e), lambda i: (0, 0)),
        ],
        out_specs=[
            pl.BlockSpec((BT, n_e), lambda i: (i, 0)),
            pl.BlockSpec((BT * c, d), lambda i: (i, 0)),
        ],
        out_shape=[
            jax.ShapeDtypeStruct((bn, n_e), jnp.float32),
            jax.ShapeDtypeStruct((bn * c, d), jnp.bfloat16),
        ],
        compiler_params=pltpu.CompilerParams(
            dimension_semantics=("arbitrary",),
        ),
    )(x2, wg, bg[None, :])


def _top2(vs):
    n = len(vs)

    def vi(c):
        return jnp.full((16,), c, jnp.int32)

    def vf(c):
        return jnp.full((16,), c, jnp.float32)

    m1 = vs[0]
    for e in range(1, n):
        m1 = jnp.maximum(m1, vs[e])
    i1 = vi(n - 1)
    for e in range(n - 2, -1, -1):
        i1 = jnp.where(vs[e] == m1, vi(e), i1)
    vs2 = [jnp.where(i1 == vi(e), vf(-1.0), vs[e]) for e in range(n)]
    m2 = vs2[0]
    for e in range(1, n):
        m2 = jnp.maximum(m2, vs2[e])
    i2 = vi(n - 1)
    for e in range(n - 2, -1, -1):
        i2 = jnp.where(vs2[e] == m2, vi(e), i2)
    return m1, i1, m2, i2


def _route_body(gate_ref, out_ref, gv, fg, cnt_v, all_cnts, shared_cnts,
                *, n_e, rows_per_sub, capacity):
    wid = lax.axis_index("s")
    base = wid * rows_per_sub
    ng = rows_per_sub // 16
    pltpu.sync_copy(gate_ref.at[:, pl.ds(base, rows_per_sub)], gv)
    iota = lax.broadcasted_iota(jnp.int32, (16,), 0)

    def vi(c):
        return jnp.full((16,), c, jnp.int32)

    def vf(c):
        return jnp.full((16,), c, jnp.float32)

    zf = jnp.zeros((16,), jnp.float32)

    def p1_body(g, cnt):
        vs = [gv[e, pl.ds(g * 16, 16)] for e in range(n_e)]
        m1, i1, m2, i2 = _top2(vs)
        for e in range(n_e):
            hot = ((i1 == vi(e)) | (i2 == vi(e))).astype(jnp.float32)
            cnt = cnt + jnp.where(iota == vi(e), vf(jnp.sum(hot)), zf)
        return cnt

    cnt = lax.fori_loop(0, ng, p1_body, zf)
    cnt_v[...] = cnt
    pltpu.sync_copy(cnt_v, shared_cnts.at[wid])
    plsc.subcore_barrier()
    pltpu.sync_copy(shared_cnts, all_cnts)

    carry0 = zf
    for w in range(16):
        scale = jnp.where(wid > w, jnp.float32(1.0), jnp.float32(0.0))
        carry0 = carry0 + all_cnts[w] * vf(scale)
    carrys = tuple(jnp.sum(jnp.where(iota == vi(e), carry0, zf))
                   for e in range(n_e))

    def p2_body(g, carrys_t):
        carrys = list(carrys_t)
        vs = [gv[e, pl.ds(g * 16, 16)] for e in range(n_e)]
        m1, i1, m2, i2 = _top2(vs)
        masked = []
        cap_v = vf(capacity)
        for e in range(n_e):
            hot = ((i1 == vi(e)) | (i2 == vi(e))).astype(jnp.float32)
            incl = plsc.cumsum(hot) + vf(carrys[e])
            carrys[e] = carrys[e] + jnp.sum(hot)
            masked.append(jnp.where(incl > cap_v, zf, vs[e]))
        n1, j1, n2, j2 = _top2(masked)
        for e in range(n_e):
            fg[e, pl.ds(g * 16, 16)] = jnp.where(
                j1 == vi(e), n1, jnp.where(j2 == vi(e), n2, zf))
        return tuple(carrys)

    lax.fori_loop(0, ng, p2_body, carrys)
    pltpu.sync_copy(fg, out_ref.at[:, pl.ds(base, rows_per_sub)])


def _route_sc(gate_t, n_e, bn, capacity):
    rows_per_sub = bn // 16
    mesh = plsc.VectorSubcoreMesh(core_axis_name="c", subcore_axis_name="s",
                                  num_cores=1)
    f = functools.partial(_route_body, n_e=n_e, rows_per_sub=rows_per_sub,
                          capacity=capacity)
    return pl.kernel(
        f,
        mesh=mesh,
        out_type=jax.ShapeDtypeStruct((n_e, bn), jnp.float32),
        scratch_types=[
            pltpu.VMEM((n_e, rows_per_sub), jnp.float32),
            pltpu.VMEM((n_e, rows_per_sub), jnp.float32),
            pltpu.VMEM((16,), jnp.float32),
            pltpu.VMEM((16, 16), jnp.float32),
            pltpu.VMEM_SHARED((16, 16), jnp.float32),
        ],
        compiler_params=pltpu.CompilerParams(needs_layout_passes=False),
    )(gate_t)


def kernel(x, Wg, bg, W1, b1, W2, b2):
    Bn, C, D = x.shape
    E = Wg.shape[1]
    gate, xbf2 = _gate_tc(x.reshape(Bn * C, D), Wg, bg, Bn, C)

    final_gate = _route_sc(gate.T, E, Bn, 2.4 * Bn / E).T

    xf = xbf2.reshape(Bn, C * D)
    out, diag = _experts_and_combine(
        xf, W1.astype(jnp.bfloat16), b1, W2.astype(jnp.bfloat16), b2,
        final_gate)
    return (out, diag)


# speedup vs baseline: 1.0481x; 1.0481x over previous
import functools

import jax
import jax.numpy as jnp
from jax import lax
from jax.experimental import pallas as pl
from jax.experimental.pallas import tpu as pltpu
from jax.experimental.pallas import tpu_sc as plsc


def _expert_body(x_ref, w1_ref, b1_ref, w2_ref, b2_ref, g_ref,
                 out_ref, diag_ref, acc_o, acc_s1, acc_s2, *, n_e):
    e = pl.program_id(1)

    h = jnp.dot(x_ref[...], w1_ref[0], preferred_element_type=jnp.float32)
    h = jnp.maximum(h + b1_ref[0], 0.0)
    ex = jnp.dot(h.astype(jnp.bfloat16), w2_ref[0],
                 preferred_element_type=jnp.float32)
    ex = ex + b2_ref[0]
    sel = jax.lax.broadcasted_iota(jnp.int32, (1, n_e), 1) == e
    g = jnp.sum(jnp.where(sel, g_ref[...], 0.0), axis=1, keepdims=True)

    @pl.when(e == 0)
    def _():
        acc_o[...] = ex * g
        acc_s1[...] = ex
        acc_s2[...] = ex * ex

    @pl.when(e > 0)
    def _():
        acc_o[...] += ex * g
        acc_s1[...] += ex
        acc_s2[...] += ex * ex

    @pl.when(e == n_e - 1)
    def _():
        out_ref[...] = acc_o[...]
        s1 = acc_s1[...]
        diag_ref[...] = (acc_s2[...] - s1 * s1 * (1.0 / n_e)) * (1.0 / (n_e - 1))


def _experts_and_combine(xf, w1bf, b1, w2bf, b2, gate, *, interpret=False):
    Bn, CD = xf.shape
    E, _, H = w1bf.shape
    K = w2bf.shape[2]
    BT = min(512, Bn)
    grid = (Bn // BT, E)
    return pl.pallas_call(
        functools.partial(_expert_body, n_e=E),
        grid=grid,
        in_specs=[
            pl.BlockSpec((BT, CD), lambda i, e: (i, 0)),
            pl.BlockSpec((1, CD, H), lambda i, e: (e, 0, 0)),
            pl.BlockSpec((1, 1, H), lambda i, e: (e, 0, 0)),
            pl.BlockSpec((1, H, K), lambda i, e: (e, 0, 0)),
            pl.BlockSpec((1, 1, K), lambda i, e: (e, 0, 0)),
            pl.BlockSpec((BT, E), lambda i, e: (i, 0)),
        ],
        out_specs=[
            pl.BlockSpec((BT, K), lambda i, e: (i, 0)),
            pl.BlockSpec((BT, K), lambda i, e: (i, 0)),
        ],
        out_shape=[
            jax.ShapeDtypeStruct((Bn, K), jnp.float32),
            jax.ShapeDtypeStruct((Bn, K), jnp.float32),
        ],
        scratch_shapes=[
            pltpu.VMEM((BT, K), jnp.float32),
            pltpu.VMEM((BT, K), jnp.float32),
            pltpu.VMEM((BT, K), jnp.float32),
        ],
        compiler_params=pltpu.CompilerParams(
            dimension_semantics=("parallel", "arbitrary"),
            vmem_limit_bytes=63 * 1024 * 1024,
        ),
        interpret=interpret,
    )(xf, w1bf, b1[:, None, :], w2bf, b2[:, None, :], gate)


def _gate_body(x_ref, wg_ref, bg_ref, out_ref, *, c):
    gl = jnp.dot(x_ref[...], wg_ref[...],
                 preferred_element_type=jnp.float32) + bg_ref[...]
    m = jnp.max(gl, axis=1, keepdims=True)
    p = jnp.exp(gl - m)
    g = p / jnp.sum(p, axis=1, keepdims=True)
    bt, n_e = out_ref.shape
    out_ref[...] = jnp.sum(g.reshape(bt, c, n_e), axis=1) * (1.0 / c)


def _gate_tc(x2, wg, bg, bn, c):
    d = x2.shape[1]
    n_e = wg.shape[1]
    BT = 256
    return pl.pallas_call(
        functools.partial(_gate_body, c=c),
        grid=(bn // BT,),
        in_specs=[
            pl.BlockSpec((BT * c, d), lambda i: (i, 0)),
            pl.BlockSpec((d, n_e), lambda i: (0, 0)),
            pl.BlockSpec((1, n_e), lambda i: (0, 0)),
        ],
        out_specs=pl.BlockSpec((BT, n_e), lambda i: (i, 0)),
        out_shape=jax.ShapeDtypeStruct((bn, n_e), jnp.float32),
        compiler_params=pltpu.CompilerParams(
            dimension_semantics=("arbitrary",),
        ),
    )(x2, wg, bg[None, :])


def _top2(vs):
    n = len(vs)

    def vi(c):
        return jnp.full((16,), c, jnp.int32)

    def vf(c):
        return jnp.full((16,), c, jnp.float32)

    m1 = vs[0]
    for e in range(1, n):
        m1 = jnp.maximum(m1, vs[e])
    i1 = vi(n - 1)
    for e in range(n - 2, -1, -1):
        i1 = jnp.where(vs[e] == m1, vi(e), i1)
    vs2 = [jnp.where(i1 == vi(e), vf(-1.0), vs[e]) for e in range(n)]
    m2 = vs2[0]
    for e in range(1, n):
        m2 = jnp.maximum(m2, vs2[e])
    i2 = vi(n - 1)
    for e in range(n - 2, -1, -1):
        i2 = jnp.where(vs2[e] == m2, vi(e), i2)
    return m1, i1, m2, i2


def _route_body(gate_ref, out_ref, gv, fg, cnt_v, all_cnts, shared_cnts,
                *, n_e, rows_per_sub, capacity):
    wid = lax.axis_index("s")
    base = wid * rows_per_sub
    ng = rows_per_sub // 16
    pltpu.sync_copy(gate_ref.at[:, pl.ds(base, rows_per_sub)], gv)
    iota = lax.broadcasted_iota(jnp.int32, (16,), 0)

    def vi(c):
        return jnp.full((16,), c, jnp.int32)

    def vf(c):
        return jnp.full((16,), c, jnp.float32)

    zf = jnp.zeros((16,), jnp.float32)

    def p1_body(g, cnt):
        vs = [gv[e, pl.ds(g * 16, 16)] for e in range(n_e)]
        m1, i1, m2, i2 = _top2(vs)
        for e in range(n_e):
            hot = ((i1 == vi(e)) | (i2 == vi(e))).astype(jnp.float32)
            cnt = cnt + jnp.where(iota == vi(e), vf(jnp.sum(hot)), zf)
        return cnt

    cnt = lax.fori_loop(0, ng, p1_body, zf)
    cnt_v[...] = cnt
    pltpu.sync_copy(cnt_v, shared_cnts.at[wid])
    plsc.subcore_barrier()
    pltpu.sync_copy(shared_cnts, all_cnts)

    carry0 = zf
    for w in range(16):
        scale = jnp.where(wid > w, jnp.float32(1.0), jnp.float32(0.0))
        carry0 = carry0 + all_cnts[w] * vf(scale)
    carrys = tuple(jnp.sum(jnp.where(iota == vi(e), carry0, zf))
                   for e in range(n_e))

    def p2_body(g, carrys_t):
        carrys = list(carrys_t)
        vs = [gv[e, pl.ds(g * 16, 16)] for e in range(n_e)]
        m1, i1, m2, i2 = _top2(vs)
        masked = []
        cap_v = vf(capacity)
        for e in range(n_e):
            hot = ((i1 == vi(e)) | (i2 == vi(e))).astype(jnp.float32)
            incl = plsc.cumsum(hot) + vf(carrys[e])
            carrys[e] = carrys[e] + jnp.sum(hot)
            masked.append(jnp.where(incl > cap_v, zf, vs[e]))
        n1, j1, n2, j2 = _top2(masked)
        for e in range(n_e):
            fg[e, pl.ds(g * 16, 16)] = jnp.where(
                j1 == vi(e), n1, jnp.where(j2 == vi(e), n2, zf))
        return tuple(carrys)

    lax.fori_loop(0, ng, p2_body, carrys)
    pltpu.sync_copy(fg, out_ref.at[:, pl.ds(base, rows_per_sub)])


def _route_sc(gate_t, n_e, bn, capacity):
    rows_per_sub = bn // 16
    mesh = plsc.VectorSubcoreMesh(core_axis_name="c", subcore_axis_name="s",
                                  num_cores=1)
    f = functools.partial(_route_body, n_e=n_e, rows_per_sub=rows_per_sub,
                          capacity=capacity)
    return pl.kernel(
        f,
        mesh=mesh,
        out_type=jax.ShapeDtypeStruct((n_e, bn), jnp.float32),
        scratch_types=[
            pltpu.VMEM((n_e, rows_per_sub), jnp.float32),
            pltpu.VMEM((n_e, rows_per_sub), jnp.float32),
            pltpu.VMEM((16,), jnp.float32),
            pltpu.VMEM((16, 16), jnp.float32),
            pltpu.VMEM_SHARED((16, 16), jnp.float32),
        ],
        compiler_params=pltpu.CompilerParams(needs_layout_passes=False),
    )(gate_t)


def kernel(x, Wg, bg, W1, b1, W2, b2):
    Bn, C, D = x.shape
    E = Wg.shape[1]
    gate = _gate_tc(x.reshape(Bn * C, D), Wg, bg, Bn, C)

    final_gate = _route_sc(gate.T, E, Bn, 2.4 * Bn / E).T

    xf = x.reshape(Bn, C * D).astype(jnp.bfloat16)
    out, diag = _experts_and_combine(
        xf, W1.astype(jnp.bfloat16), b1, W2.astype(jnp.bfloat16), b2,
        final_gate)
    return (out, diag)
